# Initial kernel scaffold; baseline (speedup 1.0000x reference)
import jax
import jax.numpy as jnp
from jax.experimental import pallas as pl


def kernel(x, edge_index, bn_g1, bn_b1, Wsrc1, bsrc1, Wfc1, bfc1, Wdst1, bdst1, bn_g2, bn_b2, Wsrc2, bsrc2, Wfc2, bfc2, Wdst2, bdst2, bn_g3, bn_b3, Wsrc3, bsrc3, Wfc3, bfc3, Wdst3, bdst3):
    raise NotImplementedError("write your pallas kernel here")


# speedup vs baseline: 6.5880x; 6.5880x over previous
import functools

import jax
import jax.numpy as jnp
from jax import lax
from jax.experimental import pallas as pl
from jax.experimental.pallas import tpu as pltpu
from jax.experimental.pallas import tpu_sc as plsc

N = 10000
E = 320000
D = 128
H = 32

NNP = 10016
TRASH = 10000
NW = 32
EPW = E // NW
CW = 128
CH = (EPW + CW - 1) // CW
EPW_PAD = CH * CW
RPS = NNP // 16

BR = 2000
NB = N // BR


def _sc_agg_body(y_hbm, srcs_hbm, dsts_hbm, zeros_hbm, out_hbm,
                 src_v, dst_v, rows_v, acc_sh, sem):
    cid = lax.axis_index("c")
    sid = lax.axis_index("s")
    wid = sid * 2 + cid
    r0 = sid * RPS
    pltpu.sync_copy(zeros_hbm.at[pl.ds(r0, RPS)], acc_sh.at[pl.ds(r0, RPS)])
    pltpu.sync_copy(srcs_hbm.at[wid], src_v)
    pltpu.sync_copy(dsts_hbm.at[wid], dst_v)
    plsc.subcore_barrier()

    def body(j, carry):
        pltpu.async_copy(y_hbm.at[src_v.at[j]], rows_v, sem).wait()
        pltpu.sync_copy(rows_v, acc_sh.at[dst_v.at[j]], add=True)
        return carry

    lax.fori_loop(0, CH, body, 0)
    plsc.subcore_barrier()
    pltpu.sync_copy(acc_sh.at[pl.ds(r0, RPS)],
                    out_hbm.at[cid, pl.ds(r0, RPS)])


_sc_agg = functools.partial(
    pl.kernel,
    mesh=plsc.VectorSubcoreMesh(core_axis_name="c", subcore_axis_name="s"),
    out_type=jax.ShapeDtypeStruct((2, NNP, H), jnp.float32),
    scratch_types=[
        pltpu.VMEM((CH, CW), jnp.int32),
        pltpu.VMEM((CH, CW), jnp.int32),
        pltpu.VMEM((CW, H), jnp.float32),
        pltpu.VMEM_SHARED((NNP, H), jnp.float32),
        pltpu.SemaphoreType.DMA,
    ],
    compiler_params=pltpu.CompilerParams(use_tc_tiling_on_sc=False),
)(_sc_agg_body)


_PREC = lax.Precision.HIGHEST
_FULL = lambda shape: pl.BlockSpec(shape, lambda i: tuple(0 for _ in shape))


def _tc_cnt_body(cntPd_ref, cntPs_ref, cd_ref, cs_ref):
    cd_ref[...] = cntPd_ref[0, 0:N, 0:1] + cntPd_ref[1, 0:N, 0:1]
    cs_ref[...] = cntPs_ref[0, 0:N, 0:1] + cntPs_ref[1, 0:N, 0:1]


def _tc_cnt(cntPd, cntPs):
    return pl.pallas_call(
        _tc_cnt_body,
        out_shape=(jax.ShapeDtypeStruct((N, 1), jnp.float32),
                   jax.ShapeDtypeStruct((N, 1), jnp.float32)),
    )(cntPd, cntPs)


def _stats_update(st_ref, h, cs):
    @pl.when(pl.program_id(0) == 0)
    def _():
        st_ref[...] = jnp.zeros((D, 8), jnp.float32)
    s1 = lax.dot_general(h, cs, (((0,), (0,)), ((), ())), precision=_PREC)
    s2 = lax.dot_general(h * h, cs, (((0,), (0,)), ((), ())), precision=_PREC)
    st_ref[:, 0:1] += s1
    st_ref[:, 1:2] += s2


def _tc_stats_body(x_ref, cs_ref, st_ref):
    _stats_update(st_ref, x_ref[...], cs_ref[...])


def _tc_stats(x, cs):
    return pl.pallas_call(
        _tc_stats_body,
        grid=(NB,),
        in_specs=[pl.BlockSpec((BR, D), lambda i: (i, 0)),
                  pl.BlockSpec((BR, 1), lambda i: (i, 0))],
        out_specs=pl.BlockSpec((D, 8), lambda i: (0, 0)),
        out_shape=jax.ShapeDtypeStruct((D, 8), jnp.float32),
    )(x, cs)


def _tc_msg_body(h_ref, st_ref, g_ref, b_ref, Wsrc_ref, bsrc_ref, y_ref):
    mu = st_ref[:, 0:1] * (1.0 / E)
    var = st_ref[:, 1:2] * (1.0 / E) - mu * mu
    scale = g_ref[...] * lax.rsqrt(var + 1e-5)
    shift = b_ref[...] - mu * scale
    Wsrc = Wsrc_ref[...]
    Wp = scale * Wsrc
    bp = lax.dot_general(shift, Wsrc, (((0,), (0,)), ((), ())),
                         precision=_PREC) + bsrc_ref[...]
    z = jnp.dot(h_ref[...], Wp, precision=_PREC) + bp
    y_ref[...] = z * 0.5 * (1.0 + lax.erf(z * 0.7071067811865476))


def _tc_msg(h, st, g, b, Wsrc, bsrc):
    return pl.pallas_call(
        _tc_msg_body,
        grid=(NB,),
        in_specs=[pl.BlockSpec((BR, D), lambda i: (i, 0)),
                  _FULL((D, 8)), _FULL((D, 1)), _FULL((D, 1)),
                  _FULL((D, H)), _FULL((1, H))],
        out_specs=pl.BlockSpec((BR, H), lambda i: (i, 0)),
        out_shape=jax.ShapeDtypeStruct((N, H), jnp.float32),
    )(h, st, g, b, Wsrc, bsrc)


def _tc_apply_body(apply_relu, with_stats, dout,
                   h_ref, aggP_ref, cd_ref, cs_ref,
                   Wfca_ref, Wfcb_ref, bfc_ref, Wdst_ref, bdst_ref,
                   *out_refs):
    h_in = h_ref[...]
    agg = aggP_ref[0, :, :] + aggP_ref[1, :, :]
    neigh = agg / jnp.maximum(cd_ref[...], 1.0)
    rst = (jnp.dot(h_in, Wfca_ref[...], precision=_PREC)
           + jnp.dot(neigh, Wfcb_ref[...], precision=_PREC) + bfc_ref[...])
    if apply_relu:
        rst = jnp.maximum(rst, 0.0)
    h = jnp.dot(h_in, Wdst_ref[...], precision=_PREC) + bdst_ref[...] + rst
    out_refs[0][...] = h
    if with_stats:
        _stats_update(out_refs[1], h, cs_ref[...])


def _tc_apply(apply_relu, with_stats, dout,
              h, aggP, cd, cs, Wfca, Wfcb, bfc, Wdst, bdst):
    out_shape = [jax.ShapeDtypeStruct((N, dout), jnp.float32)]
    out_specs = [pl.BlockSpec((BR, dout), lambda i: (i, 0))]
    if with_stats:
        out_shape.append(jax.ShapeDtypeStruct((D, 8), jnp.float32))
        out_specs.append(pl.BlockSpec((D, 8), lambda i: (0, 0)))
    res = pl.pallas_call(
        functools.partial(_tc_apply_body, apply_relu, with_stats, dout),
        grid=(NB,),
        in_specs=[pl.BlockSpec((BR, D), lambda i: (i, 0)),
                  pl.BlockSpec((2, BR, H), lambda i: (0, i, 0)),
                  pl.BlockSpec((BR, 1), lambda i: (i, 0)),
                  pl.BlockSpec((BR, 1), lambda i: (i, 0)),
                  _FULL((D, dout)), _FULL((H, dout)), _FULL((1, dout)),
                  _FULL((D, dout)), _FULL((1, dout))],
        out_specs=out_specs,
        out_shape=out_shape,
    )(h, aggP, cd, cs, Wfca, Wfcb, bfc, Wdst, bdst)
    return res


@jax.jit
def kernel(x, edge_index,
           bn_g1, bn_b1, Wsrc1, bsrc1, Wfc1, bfc1, Wdst1, bdst1,
           bn_g2, bn_b2, Wsrc2, bsrc2, Wfc2, bfc2, Wdst2, bdst2,
           bn_g3, bn_b3, Wsrc3, bsrc3, Wfc3, bfc3, Wdst3, bdst3):
    src = edge_index[0]
    dst = edge_index[1]
    spad = jnp.zeros((NW, EPW_PAD - EPW), jnp.int32)
    dpad = jnp.full((NW, EPW_PAD - EPW), TRASH, jnp.int32)
    srcs_r = jnp.concatenate([src.reshape(NW, EPW), spad], axis=1)
    srcs_r = srcs_r.reshape(NW, CH, CW)
    dsts_r = jnp.concatenate([dst.reshape(NW, EPW), dpad], axis=1)
    dsts_r = dsts_r.reshape(NW, CH, CW)

    zeros = jnp.zeros((NNP, H), jnp.float32)
    ones = jnp.ones((N, H), jnp.float32)

    cntPd = _sc_agg(ones, srcs_r, dsts_r, zeros)
    cntPs = _sc_agg(ones, dsts_r, srcs_r, zeros)
    cd, cs = _tc_cnt(cntPd, cntPs)

    g1 = bn_g1.reshape(D, 1); b1 = bn_b1.reshape(D, 1)
    g2 = bn_g2.reshape(D, 1); b2 = bn_b2.reshape(D, 1)
    g3 = bn_g3.reshape(D, 1); b3 = bn_b3.reshape(D, 1)

    st1 = _tc_stats(x, cs)
    y1 = _tc_msg(x, st1, g1, b1, Wsrc1, bsrc1.reshape(1, H))
    aggP1 = _sc_agg(y1, srcs_r, dsts_r, zeros)
    h1, st2 = _tc_apply(True, True, D, x, aggP1, cd, cs,
                        Wfc1[0:D], Wfc1[D:], bfc1.reshape(1, D),
                        Wdst1, bdst1.reshape(1, D))
    y2 = _tc_msg(h1, st2, g2, b2, Wsrc2, bsrc2.reshape(1, H))
    aggP2 = _sc_agg(y2, srcs_r, dsts_r, zeros)
    h2, st3 = _tc_apply(False, True, D, h1, aggP2, cd, cs,
                        Wfc2[0:D], Wfc2[D:], bfc2.reshape(1, D),
                        Wdst2, bdst2.reshape(1, D))
    y3 = _tc_msg(h2, st3, g3, b3, Wsrc3, bsrc3.reshape(1, H))
    aggP3 = _sc_agg(y3, srcs_r, dsts_r, zeros)
    (out,) = _tc_apply(False, False, 1, h2, aggP3, cd, cs,
                       Wfc3[0:D], Wfc3[D:], bfc3.reshape(1, 1),
                       Wdst3, bdst3.reshape(1, 1))
    return out


# speedup vs baseline: 8.3222x; 1.2632x over previous
import functools

import jax
import jax.numpy as jnp
from jax import lax
from jax.experimental import pallas as pl
from jax.experimental.pallas import tpu as pltpu
from jax.experimental.pallas import tpu_sc as plsc

N = 10000
E = 320000
D = 128
H = 32

NNP = 10016
TRASH = 10000
NW = 32
EPW = E // NW
CW = 128
CH = 80
EPW_PAD = CH * CW
RPS = NNP // 16

BR = 2000
NB = N // BR


def _sc_agg_body(y_hbm, srcs_hbm, dsts_hbm, zeros_hbm, out_hbm,
                 src_v, dst_v, rows0_v, rows1_v, acc_sh, sem0, sem1):
    cid = lax.axis_index("c")
    sid = lax.axis_index("s")
    wid = sid * 2 + cid
    r0 = sid * RPS
    pltpu.sync_copy(zeros_hbm.at[pl.ds(r0, RPS)], acc_sh.at[pl.ds(r0, RPS)])
    pltpu.sync_copy(srcs_hbm.at[wid], src_v)
    pltpu.sync_copy(dsts_hbm.at[wid], dst_v)
    plsc.subcore_barrier()

    g0 = pltpu.async_copy(y_hbm.at[src_v.at[0]], rows0_v, sem0)

    def body(t, carry):
        c0 = 2 * t
        pltpu.async_copy(y_hbm.at[src_v.at[c0 + 1]], rows1_v, sem1)
        pltpu.make_async_copy(y_hbm.at[src_v.at[c0]], rows0_v, sem0).wait()
        pltpu.sync_copy(rows0_v, acc_sh.at[dst_v.at[c0]], add=True)

        @pl.when(t < CH // 2 - 1)
        def _():
            pltpu.async_copy(y_hbm.at[src_v.at[c0 + 2]], rows0_v, sem0)

        pltpu.make_async_copy(y_hbm.at[src_v.at[c0 + 1]], rows1_v,
                              sem1).wait()
        pltpu.sync_copy(rows1_v, acc_sh.at[dst_v.at[c0 + 1]], add=True)
        return carry

    lax.fori_loop(0, CH // 2, body, 0)
    plsc.subcore_barrier()
    pltpu.sync_copy(acc_sh.at[pl.ds(r0, RPS)],
                    out_hbm.at[cid, pl.ds(r0, RPS)])


_sc_agg = functools.partial(
    pl.kernel,
    mesh=plsc.VectorSubcoreMesh(core_axis_name="c", subcore_axis_name="s"),
    out_type=jax.ShapeDtypeStruct((2, NNP, H), jnp.float32),
    scratch_types=[
        pltpu.VMEM((CH, CW), jnp.int32),
        pltpu.VMEM((CH, CW), jnp.int32),
        pltpu.VMEM((CW, H), jnp.float32),
        pltpu.VMEM((CW, H), jnp.float32),
        pltpu.VMEM_SHARED((NNP, H), jnp.float32),
        pltpu.SemaphoreType.DMA,
        pltpu.SemaphoreType.DMA,
    ],
    compiler_params=pltpu.CompilerParams(use_tc_tiling_on_sc=False),
)(_sc_agg_body)


def _sc_cnt_body(ones_hbm, srcs_hbm, dsts_hbm, zeros_hbm, out_hbm,
                 src_v, dst_v, ones_v, accd_sh, accs_sh, sem):
    cid = lax.axis_index("c")
    sid = lax.axis_index("s")
    wid = sid * 2 + cid
    r0 = sid * RPS
    pltpu.sync_copy(zeros_hbm.at[pl.ds(r0, RPS)], accd_sh.at[pl.ds(r0, RPS)])
    pltpu.sync_copy(zeros_hbm.at[pl.ds(r0, RPS)], accs_sh.at[pl.ds(r0, RPS)])
    pltpu.sync_copy(srcs_hbm.at[wid], src_v)
    pltpu.sync_copy(dsts_hbm.at[wid], dst_v)
    pltpu.sync_copy(ones_hbm, ones_v)
    plsc.subcore_barrier()

    def body(j, carry):
        d1 = pltpu.async_copy(ones_v, accd_sh.at[dst_v.at[j]], sem, add=True)
        d2 = pltpu.async_copy(ones_v, accs_sh.at[src_v.at[j]], sem, add=True)
        d1.wait()
        d2.wait()
        return carry

    lax.fori_loop(0, CH, body, 0)
    plsc.subcore_barrier()
    pltpu.sync_copy(accd_sh.at[pl.ds(r0, RPS)],
                    out_hbm.at[cid, 0, pl.ds(r0, RPS)])
    pltpu.sync_copy(accs_sh.at[pl.ds(r0, RPS)],
                    out_hbm.at[cid, 1, pl.ds(r0, RPS)])


_sc_cnt = functools.partial(
    pl.kernel,
    mesh=plsc.VectorSubcoreMesh(core_axis_name="c", subcore_axis_name="s"),
    out_type=jax.ShapeDtypeStruct((2, 2, NNP, H), jnp.float32),
    scratch_types=[
        pltpu.VMEM((CH, CW), jnp.int32),
        pltpu.VMEM((CH, CW), jnp.int32),
        pltpu.VMEM((CW, H), jnp.float32),
        pltpu.VMEM_SHARED((NNP, H), jnp.float32),
        pltpu.VMEM_SHARED((NNP, H), jnp.float32),
        pltpu.SemaphoreType.DMA,
    ],
    compiler_params=pltpu.CompilerParams(use_tc_tiling_on_sc=False),
)(_sc_cnt_body)


_PREC = lax.Precision.HIGHEST
_FULL = lambda shape: pl.BlockSpec(shape, lambda i: tuple(0 for _ in shape))


def _tc_cnt_body(cntP_ref, cd_ref, cs_ref):
    cd_ref[...] = cntP_ref[0, 0, 0:N, 0:1] + cntP_ref[1, 0, 0:N, 0:1]
    cs_ref[...] = cntP_ref[0, 1, 0:N, 0:1] + cntP_ref[1, 1, 0:N, 0:1]


def _tc_cnt(cntP):
    return pl.pallas_call(
        _tc_cnt_body,
        out_shape=(jax.ShapeDtypeStruct((N, 1), jnp.float32),
                   jax.ShapeDtypeStruct((N, 1), jnp.float32)),
    )(cntP)


def _stats_update(st_ref, h, cs):
    @pl.when(pl.program_id(0) == 0)
    def _():
        st_ref[...] = jnp.zeros((D, 8), jnp.float32)
    s1 = lax.dot_general(h, cs, (((0,), (0,)), ((), ())), precision=_PREC)
    s2 = lax.dot_general(h * h, cs, (((0,), (0,)), ((), ())), precision=_PREC)
    st_ref[:, 0:1] += s1
    st_ref[:, 1:2] += s2


def _tc_stats_body(x_ref, cs_ref, st_ref):
    _stats_update(st_ref, x_ref[...], cs_ref[...])


def _tc_stats(x, cs):
    return pl.pallas_call(
        _tc_stats_body,
        grid=(NB,),
        in_specs=[pl.BlockSpec((BR, D), lambda i: (i, 0)),
                  pl.BlockSpec((BR, 1), lambda i: (i, 0))],
        out_specs=pl.BlockSpec((D, 8), lambda i: (0, 0)),
        out_shape=jax.ShapeDtypeStruct((D, 8), jnp.float32),
    )(x, cs)


def _tc_msg_body(h_ref, st_ref, g_ref, b_ref, Wsrc_ref, bsrc_ref, y_ref):
    mu = st_ref[:, 0:1] * (1.0 / E)
    var = st_ref[:, 1:2] * (1.0 / E) - mu * mu
    scale = g_ref[...] * lax.rsqrt(var + 1e-5)
    shift = b_ref[...] - mu * scale
    Wsrc = Wsrc_ref[...]
    Wp = scale * Wsrc
    bp = lax.dot_general(shift, Wsrc, (((0,), (0,)), ((), ())),
                         precision=_PREC) + bsrc_ref[...]
    z = jnp.dot(h_ref[...], Wp, precision=_PREC) + bp
    y_ref[...] = z * 0.5 * (1.0 + lax.erf(z * 0.7071067811865476))


def _tc_msg(h, st, g, b, Wsrc, bsrc):
    return pl.pallas_call(
        _tc_msg_body,
        grid=(NB,),
        in_specs=[pl.BlockSpec((BR, D), lambda i: (i, 0)),
                  _FULL((D, 8)), _FULL((D, 1)), _FULL((D, 1)),
                  _FULL((D, H)), _FULL((1, H))],
        out_specs=pl.BlockSpec((BR, H), lambda i: (i, 0)),
        out_shape=jax.ShapeDtypeStruct((N, H), jnp.float32),
    )(h, st, g, b, Wsrc, bsrc)


def _tc_apply_body(apply_relu, with_stats, dout,
                   h_ref, aggP_ref, cd_ref, cs_ref,
                   Wfca_ref, Wfcb_ref, bfc_ref, Wdst_ref, bdst_ref,
                   *out_refs):
    h_in = h_ref[...]
    agg = aggP_ref[0, :, :] + aggP_ref[1, :, :]
    neigh = agg / jnp.maximum(cd_ref[...], 1.0)
    rst = (jnp.dot(h_in, Wfca_ref[...], precision=_PREC)
           + jnp.dot(neigh, Wfcb_ref[...], precision=_PREC) + bfc_ref[...])
    if apply_relu:
        rst = jnp.maximum(rst, 0.0)
    h = jnp.dot(h_in, Wdst_ref[...], precision=_PREC) + bdst_ref[...] + rst
    out_refs[0][...] = h
    if with_stats:
        _stats_update(out_refs[1], h, cs_ref[...])


def _tc_apply(apply_relu, with_stats, dout,
              h, aggP, cd, cs, Wfca, Wfcb, bfc, Wdst, bdst):
    out_shape = [jax.ShapeDtypeStruct((N, dout), jnp.float32)]
    out_specs = [pl.BlockSpec((BR, dout), lambda i: (i, 0))]
    if with_stats:
        out_shape.append(jax.ShapeDtypeStruct((D, 8), jnp.float32))
        out_specs.append(pl.BlockSpec((D, 8), lambda i: (0, 0)))
    res = pl.pallas_call(
        functools.partial(_tc_apply_body, apply_relu, with_stats, dout),
        grid=(NB,),
        in_specs=[pl.BlockSpec((BR, D), lambda i: (i, 0)),
                  pl.BlockSpec((2, BR, H), lambda i: (0, i, 0)),
                  pl.BlockSpec((BR, 1), lambda i: (i, 0)),
                  pl.BlockSpec((BR, 1), lambda i: (i, 0)),
                  _FULL((D, dout)), _FULL((H, dout)), _FULL((1, dout)),
                  _FULL((D, dout)), _FULL((1, dout))],
        out_specs=out_specs,
        out_shape=out_shape,
    )(h, aggP, cd, cs, Wfca, Wfcb, bfc, Wdst, bdst)
    return res


@jax.jit
def kernel(x, edge_index,
           bn_g1, bn_b1, Wsrc1, bsrc1, Wfc1, bfc1, Wdst1, bdst1,
           bn_g2, bn_b2, Wsrc2, bsrc2, Wfc2, bfc2, Wdst2, bdst2,
           bn_g3, bn_b3, Wsrc3, bsrc3, Wfc3, bfc3, Wdst3, bdst3):
    src = edge_index[0]
    dst = edge_index[1]
    spad = jnp.zeros((NW, EPW_PAD - EPW), jnp.int32)
    tpad = jnp.full((NW, EPW_PAD - EPW), TRASH, jnp.int32)
    srcs_g = jnp.concatenate([src.reshape(NW, EPW), spad], axis=1)
    srcs_g = srcs_g.reshape(NW, CH, CW)
    srcs_c = jnp.concatenate([src.reshape(NW, EPW), tpad], axis=1)
    srcs_c = srcs_c.reshape(NW, CH, CW)
    dsts_r = jnp.concatenate([dst.reshape(NW, EPW), tpad], axis=1)
    dsts_r = dsts_r.reshape(NW, CH, CW)

    zeros = jnp.zeros((NNP, H), jnp.float32)
    ones_rows = jnp.ones((CW, H), jnp.float32)

    cntP = _sc_cnt(ones_rows, srcs_c, dsts_r, zeros)
    cd, cs = _tc_cnt(cntP)

    g1 = bn_g1.reshape(D, 1); b1 = bn_b1.reshape(D, 1)
    g2 = bn_g2.reshape(D, 1); b2 = bn_b2.reshape(D, 1)
    g3 = bn_g3.reshape(D, 1); b3 = bn_b3.reshape(D, 1)

    st1 = _tc_stats(x, cs)
    y1 = _tc_msg(x, st1, g1, b1, Wsrc1, bsrc1.reshape(1, H))
    aggP1 = _sc_agg(y1, srcs_g, dsts_r, zeros)
    h1, st2 = _tc_apply(True, True, D, x, aggP1, cd, cs,
                        Wfc1[0:D], Wfc1[D:], bfc1.reshape(1, D),
                        Wdst1, bdst1.reshape(1, D))
    y2 = _tc_msg(h1, st2, g2, b2, Wsrc2, bsrc2.reshape(1, H))
    aggP2 = _sc_agg(y2, srcs_g, dsts_r, zeros)
    h2, st3 = _tc_apply(False, True, D, h1, aggP2, cd, cs,
                        Wfc2[0:D], Wfc2[D:], bfc2.reshape(1, D),
                        Wdst2, bdst2.reshape(1, D))
    y3 = _tc_msg(h2, st3, g3, b3, Wsrc3, bsrc3.reshape(1, H))
    aggP3 = _sc_agg(y3, srcs_g, dsts_r, zeros)
    (out,) = _tc_apply(False, False, 1, h2, aggP3, cd, cs,
                       Wfc3[0:D], Wfc3[D:], bfc3.reshape(1, 1),
                       Wdst3, bdst3.reshape(1, 1))
    return out


# speedup vs baseline: 12.0602x; 1.4492x over previous
import functools

import jax
import jax.numpy as jnp
from jax import lax
from jax.experimental import pallas as pl
from jax.experimental.pallas import tpu as pltpu
from jax.experimental.pallas import tpu_sc as plsc

N = 10000
E = 320000
D = 128
H = 32

NNP = 10016
TRASH = 10000
NW = 32
EPW = E // NW
CW = 128
CH = 80
EPW_PAD = CH * CW
RPS = NNP // 16

BR = 2000
NB = N // BR


YRS = N // 16


def _sc_agg_body(y_hbm, srcs_hbm, dsts_hbm, zeros_hbm, out_hbm,
                 src_v, dst_v, rows_v, y_sh, acc_sh):
    cid = lax.axis_index("c")
    sid = lax.axis_index("s")
    wid = sid * 2 + cid
    r0 = sid * RPS
    pltpu.sync_copy(zeros_hbm.at[pl.ds(r0, RPS)], acc_sh.at[pl.ds(r0, RPS)])
    pltpu.sync_copy(y_hbm.at[pl.ds(sid * YRS, YRS)],
                    y_sh.at[pl.ds(sid * YRS, YRS)])
    pltpu.sync_copy(srcs_hbm.at[wid], src_v)
    pltpu.sync_copy(dsts_hbm.at[wid], dst_v)
    plsc.subcore_barrier()

    def body(j, carry):
        pltpu.sync_copy(y_sh.at[src_v.at[j]], rows_v)
        pltpu.sync_copy(rows_v, acc_sh.at[dst_v.at[j]], add=True)
        return carry

    lax.fori_loop(0, CH, body, 0)
    plsc.subcore_barrier()
    pltpu.sync_copy(acc_sh.at[pl.ds(r0, RPS)],
                    out_hbm.at[cid, pl.ds(r0, RPS)])


_sc_agg = functools.partial(
    pl.kernel,
    mesh=plsc.VectorSubcoreMesh(core_axis_name="c", subcore_axis_name="s"),
    out_type=jax.ShapeDtypeStruct((2, NNP, H), jnp.float32),
    scratch_types=[
        pltpu.VMEM((CH, CW), jnp.int32),
        pltpu.VMEM((CH, CW), jnp.int32),
        pltpu.VMEM((CW, H), jnp.float32),
        pltpu.VMEM_SHARED((NNP, H), jnp.float32),
        pltpu.VMEM_SHARED((NNP, H), jnp.float32),
    ],
    compiler_params=pltpu.CompilerParams(use_tc_tiling_on_sc=False),
)(_sc_agg_body)


def _sc_cnt_body(ones_hbm, srcs_hbm, dsts_hbm, zeros_hbm, out_hbm,
                 src_v, dst_v, ones_v, accd_sh, accs_sh, sem):
    cid = lax.axis_index("c")
    sid = lax.axis_index("s")
    wid = sid * 2 + cid
    r0 = sid * RPS
    pltpu.sync_copy(zeros_hbm.at[pl.ds(r0, RPS)], accd_sh.at[pl.ds(r0, RPS)])
    pltpu.sync_copy(zeros_hbm.at[pl.ds(r0, RPS)], accs_sh.at[pl.ds(r0, RPS)])
    pltpu.sync_copy(srcs_hbm.at[wid], src_v)
    pltpu.sync_copy(dsts_hbm.at[wid], dst_v)
    pltpu.sync_copy(ones_hbm, ones_v)
    plsc.subcore_barrier()

    def body(j, carry):
        d1 = pltpu.async_copy(ones_v, accd_sh.at[dst_v.at[j]], sem, add=True)
        d2 = pltpu.async_copy(ones_v, accs_sh.at[src_v.at[j]], sem, add=True)
        d1.wait()
        d2.wait()
        return carry

    lax.fori_loop(0, CH, body, 0)
    plsc.subcore_barrier()
    pltpu.sync_copy(accd_sh.at[pl.ds(r0, RPS)],
                    out_hbm.at[cid, 0, pl.ds(r0, RPS)])
    pltpu.sync_copy(accs_sh.at[pl.ds(r0, RPS)],
                    out_hbm.at[cid, 1, pl.ds(r0, RPS)])


_sc_cnt = functools.partial(
    pl.kernel,
    mesh=plsc.VectorSubcoreMesh(core_axis_name="c", subcore_axis_name="s"),
    out_type=jax.ShapeDtypeStruct((2, 2, NNP, H), jnp.float32),
    scratch_types=[
        pltpu.VMEM((CH, CW), jnp.int32),
        pltpu.VMEM((CH, CW), jnp.int32),
        pltpu.VMEM((CW, H), jnp.float32),
        pltpu.VMEM_SHARED((NNP, H), jnp.float32),
        pltpu.VMEM_SHARED((NNP, H), jnp.float32),
        pltpu.SemaphoreType.DMA,
    ],
    compiler_params=pltpu.CompilerParams(use_tc_tiling_on_sc=False),
)(_sc_cnt_body)


_PREC = lax.Precision.HIGHEST
_FULL = lambda shape: pl.BlockSpec(shape, lambda i: tuple(0 for _ in shape))


def _tc_cnt_body(cntP_ref, cd_ref, cs_ref):
    cd_ref[...] = cntP_ref[0, 0, 0:N, 0:1] + cntP_ref[1, 0, 0:N, 0:1]
    cs_ref[...] = cntP_ref[0, 1, 0:N, 0:1] + cntP_ref[1, 1, 0:N, 0:1]


def _tc_cnt(cntP):
    return pl.pallas_call(
        _tc_cnt_body,
        out_shape=(jax.ShapeDtypeStruct((N, 1), jnp.float32),
                   jax.ShapeDtypeStruct((N, 1), jnp.float32)),
    )(cntP)


def _stats_update(st_ref, h, cs):
    @pl.when(pl.program_id(0) == 0)
    def _():
        st_ref[...] = jnp.zeros((D, 8), jnp.float32)
    s1 = lax.dot_general(h, cs, (((0,), (0,)), ((), ())), precision=_PREC)
    s2 = lax.dot_general(h * h, cs, (((0,), (0,)), ((), ())), precision=_PREC)
    st_ref[:, 0:1] += s1
    st_ref[:, 1:2] += s2


def _tc_stats_body(x_ref, cs_ref, st_ref):
    _stats_update(st_ref, x_ref[...], cs_ref[...])


def _tc_stats(x, cs):
    return pl.pallas_call(
        _tc_stats_body,
        grid=(NB,),
        in_specs=[pl.BlockSpec((BR, D), lambda i: (i, 0)),
                  pl.BlockSpec((BR, 1), lambda i: (i, 0))],
        out_specs=pl.BlockSpec((D, 8), lambda i: (0, 0)),
        out_shape=jax.ShapeDtypeStruct((D, 8), jnp.float32),
    )(x, cs)


def _tc_msg_body(h_ref, st_ref, g_ref, b_ref, Wsrc_ref, bsrc_ref, y_ref):
    mu = st_ref[:, 0:1] * (1.0 / E)
    var = st_ref[:, 1:2] * (1.0 / E) - mu * mu
    scale = g_ref[...] * lax.rsqrt(var + 1e-5)
    shift = b_ref[...] - mu * scale
    Wsrc = Wsrc_ref[...]
    Wp = scale * Wsrc
    bp = lax.dot_general(shift, Wsrc, (((0,), (0,)), ((), ())),
                         precision=_PREC) + bsrc_ref[...]
    z = jnp.dot(h_ref[...], Wp, precision=_PREC) + bp
    y_ref[...] = z * 0.5 * (1.0 + lax.erf(z * 0.7071067811865476))


def _tc_msg(h, st, g, b, Wsrc, bsrc):
    return pl.pallas_call(
        _tc_msg_body,
        grid=(NB,),
        in_specs=[pl.BlockSpec((BR, D), lambda i: (i, 0)),
                  _FULL((D, 8)), _FULL((D, 1)), _FULL((D, 1)),
                  _FULL((D, H)), _FULL((1, H))],
        out_specs=pl.BlockSpec((BR, H), lambda i: (i, 0)),
        out_shape=jax.ShapeDtypeStruct((N, H), jnp.float32),
    )(h, st, g, b, Wsrc, bsrc)


def _tc_apply_body(apply_relu, with_stats, dout,
                   h_ref, aggP_ref, cd_ref, cs_ref,
                   Wfca_ref, Wfcb_ref, bfc_ref, Wdst_ref, bdst_ref,
                   *out_refs):
    h_in = h_ref[...]
    agg = aggP_ref[0, :, :] + aggP_ref[1, :, :]
    neigh = agg / jnp.maximum(cd_ref[...], 1.0)
    rst = (jnp.dot(h_in, Wfca_ref[...], precision=_PREC)
           + jnp.dot(neigh, Wfcb_ref[...], precision=_PREC) + bfc_ref[...])
    if apply_relu:
        rst = jnp.maximum(rst, 0.0)
    h = jnp.dot(h_in, Wdst_ref[...], precision=_PREC) + bdst_ref[...] + rst
    out_refs[0][...] = h
    if with_stats:
        _stats_update(out_refs[1], h, cs_ref[...])


def _tc_apply(apply_relu, with_stats, dout,
              h, aggP, cd, cs, Wfca, Wfcb, bfc, Wdst, bdst):
    out_shape = [jax.ShapeDtypeStruct((N, dout), jnp.float32)]
    out_specs = [pl.BlockSpec((BR, dout), lambda i: (i, 0))]
    if with_stats:
        out_shape.append(jax.ShapeDtypeStruct((D, 8), jnp.float32))
        out_specs.append(pl.BlockSpec((D, 8), lambda i: (0, 0)))
    res = pl.pallas_call(
        functools.partial(_tc_apply_body, apply_relu, with_stats, dout),
        grid=(NB,),
        in_specs=[pl.BlockSpec((BR, D), lambda i: (i, 0)),
                  pl.BlockSpec((2, BR, H), lambda i: (0, i, 0)),
                  pl.BlockSpec((BR, 1), lambda i: (i, 0)),
                  pl.BlockSpec((BR, 1), lambda i: (i, 0)),
                  _FULL((D, dout)), _FULL((H, dout)), _FULL((1, dout)),
                  _FULL((D, dout)), _FULL((1, dout))],
        out_specs=out_specs,
        out_shape=out_shape,
    )(h, aggP, cd, cs, Wfca, Wfcb, bfc, Wdst, bdst)
    return res


@jax.jit
def kernel(x, edge_index,
           bn_g1, bn_b1, Wsrc1, bsrc1, Wfc1, bfc1, Wdst1, bdst1,
           bn_g2, bn_b2, Wsrc2, bsrc2, Wfc2, bfc2, Wdst2, bdst2,
           bn_g3, bn_b3, Wsrc3, bsrc3, Wfc3, bfc3, Wdst3, bdst3):
    src = edge_index[0]
    dst = edge_index[1]
    pspread = jnp.arange(EPW_PAD - EPW, dtype=jnp.int32)[None, :] % 16
    spad = pspread
    tpad = TRASH + pspread
    spad = jnp.broadcast_to(spad, (NW, EPW_PAD - EPW))
    tpad = jnp.broadcast_to(tpad, (NW, EPW_PAD - EPW))
    srcs_g = jnp.concatenate([src.reshape(NW, EPW), spad], axis=1)
    srcs_g = srcs_g.reshape(NW, CH, CW)
    srcs_c = jnp.concatenate([src.reshape(NW, EPW), tpad], axis=1)
    srcs_c = srcs_c.reshape(NW, CH, CW)
    dsts_r = jnp.concatenate([dst.reshape(NW, EPW), tpad], axis=1)
    dsts_r = dsts_r.reshape(NW, CH, CW)

    zeros = jnp.zeros((NNP, H), jnp.float32)
    ones_rows = jnp.ones((CW, H), jnp.float32)

    cntP = _sc_cnt(ones_rows, srcs_c, dsts_r, zeros)
    cd, cs = _tc_cnt(cntP)

    g1 = bn_g1.reshape(D, 1); b1 = bn_b1.reshape(D, 1)
    g2 = bn_g2.reshape(D, 1); b2 = bn_b2.reshape(D, 1)
    g3 = bn_g3.reshape(D, 1); b3 = bn_b3.reshape(D, 1)

    st1 = _tc_stats(x, cs)
    y1 = _tc_msg(x, st1, g1, b1, Wsrc1, bsrc1.reshape(1, H))
    aggP1 = _sc_agg(y1, srcs_g, dsts_r, zeros)
    h1, st2 = _tc_apply(True, True, D, x, aggP1, cd, cs,
                        Wfc1[0:D], Wfc1[D:], bfc1.reshape(1, D),
                        Wdst1, bdst1.reshape(1, D))
    y2 = _tc_msg(h1, st2, g2, b2, Wsrc2, bsrc2.reshape(1, H))
    aggP2 = _sc_agg(y2, srcs_g, dsts_r, zeros)
    h2, st3 = _tc_apply(False, True, D, h1, aggP2, cd, cs,
                        Wfc2[0:D], Wfc2[D:], bfc2.reshape(1, D),
                        Wdst2, bdst2.reshape(1, D))
    y3 = _tc_msg(h2, st3, g3, b3, Wsrc3, bsrc3.reshape(1, H))
    aggP3 = _sc_agg(y3, srcs_g, dsts_r, zeros)
    (out,) = _tc_apply(False, False, 1, h2, aggP3, cd, cs,
                       Wfc3[0:D], Wfc3[D:], bfc3.reshape(1, 1),
                       Wdst3, bdst3.reshape(1, 1))
    return out


# speedup vs baseline: 12.8365x; 1.0644x over previous
import functools

import jax
import jax.numpy as jnp
from jax import lax
from jax.experimental import pallas as pl
from jax.experimental.pallas import tpu as pltpu
from jax.experimental.pallas import tpu_sc as plsc

N = 10000
E = 320000
D = 128
H = 32

NNP = 10016
TRASH = 10000
NW = 32
EPW = E // NW
CW = 128
CH = 80
EPW_PAD = CH * CW
RPS = NNP // 16

BR = 2000
NB = N // BR


YRS = N // 16


def _sc_agg_body(y_hbm, srcs_hbm, dsts_hbm, zeros_hbm, out_hbm,
                 src_v, dst_v, rows_v, y_sh, acc_sh):
    cid = lax.axis_index("c")
    sid = lax.axis_index("s")
    wid = sid * 2 + cid
    r0 = sid * RPS
    pltpu.sync_copy(zeros_hbm.at[pl.ds(r0, RPS)], acc_sh.at[pl.ds(r0, RPS)])
    pltpu.sync_copy(y_hbm.at[pl.ds(sid * YRS, YRS)],
                    y_sh.at[pl.ds(sid * YRS, YRS)])
    pltpu.sync_copy(srcs_hbm.at[wid], src_v)
    pltpu.sync_copy(dsts_hbm.at[wid], dst_v)
    plsc.subcore_barrier()

    def body(j, carry):
        pltpu.sync_copy(y_sh.at[src_v.at[j]], rows_v)
        pltpu.sync_copy(rows_v, acc_sh.at[dst_v.at[j]], add=True)
        return carry

    lax.fori_loop(0, CH, body, 0)
    plsc.subcore_barrier()
    pltpu.sync_copy(acc_sh.at[pl.ds(r0, RPS)],
                    out_hbm.at[cid, pl.ds(r0, RPS)])


_sc_agg = functools.partial(
    pl.kernel,
    mesh=plsc.VectorSubcoreMesh(core_axis_name="c", subcore_axis_name="s"),
    out_type=jax.ShapeDtypeStruct((2, NNP, H), jnp.float32),
    scratch_types=[
        pltpu.VMEM((CH, CW), jnp.int32),
        pltpu.VMEM((CH, CW), jnp.int32),
        pltpu.VMEM((CW, H), jnp.float32),
        pltpu.VMEM_SHARED((NNP, H), jnp.float32),
        pltpu.VMEM_SHARED((NNP, H), jnp.float32),
    ],
    compiler_params=pltpu.CompilerParams(use_tc_tiling_on_sc=False),
)(_sc_agg_body)


HC = 8


def _sc_cnt_body(ones_hbm, srcs_hbm, dsts_hbm, zeros_hbm, out_hbm,
                 src_v, dst_v, ones_v, accd_sh, accs_sh, sem):
    cid = lax.axis_index("c")
    sid = lax.axis_index("s")
    wid = sid * 2 + cid
    r0 = sid * RPS
    pltpu.sync_copy(zeros_hbm.at[pl.ds(r0, RPS)], accd_sh.at[pl.ds(r0, RPS)])
    pltpu.sync_copy(zeros_hbm.at[pl.ds(r0, RPS)], accs_sh.at[pl.ds(r0, RPS)])
    pltpu.sync_copy(srcs_hbm.at[wid], src_v)
    pltpu.sync_copy(dsts_hbm.at[wid], dst_v)
    pltpu.sync_copy(ones_hbm, ones_v)
    plsc.subcore_barrier()

    def body(j, carry):
        d1 = pltpu.async_copy(ones_v, accd_sh.at[dst_v.at[j]], sem, add=True)
        d2 = pltpu.async_copy(ones_v, accs_sh.at[src_v.at[j]], sem, add=True)
        d1.wait()
        d2.wait()
        return carry

    lax.fori_loop(0, CH, body, 0)
    plsc.subcore_barrier()
    pltpu.sync_copy(accd_sh.at[pl.ds(r0, RPS)],
                    out_hbm.at[cid, 0, pl.ds(r0, RPS)])
    pltpu.sync_copy(accs_sh.at[pl.ds(r0, RPS)],
                    out_hbm.at[cid, 1, pl.ds(r0, RPS)])


_sc_cnt = functools.partial(
    pl.kernel,
    mesh=plsc.VectorSubcoreMesh(core_axis_name="c", subcore_axis_name="s"),
    out_type=jax.ShapeDtypeStruct((2, 2, NNP, HC), jnp.float32),
    scratch_types=[
        pltpu.VMEM((CH, CW), jnp.int32),
        pltpu.VMEM((CH, CW), jnp.int32),
        pltpu.VMEM((CW, HC), jnp.float32),
        pltpu.VMEM_SHARED((NNP, HC), jnp.float32),
        pltpu.VMEM_SHARED((NNP, HC), jnp.float32),
        pltpu.SemaphoreType.DMA,
    ],
    compiler_params=pltpu.CompilerParams(use_tc_tiling_on_sc=False),
)(_sc_cnt_body)


_PREC = lax.Precision.HIGHEST
_FULL = lambda shape: pl.BlockSpec(shape, lambda i: tuple(0 for _ in shape))


def _stats_update(st_ref, h, cs):
    @pl.when(pl.program_id(0) == 0)
    def _():
        st_ref[...] = jnp.zeros((D, 8), jnp.float32)
    s1 = lax.dot_general(h, cs, (((0,), (0,)), ((), ())), precision=_PREC)
    s2 = lax.dot_general(h * h, cs, (((0,), (0,)), ((), ())), precision=_PREC)
    st_ref[:, 0:1] += s1
    st_ref[:, 1:2] += s2


def _tc_stats_body(x_ref, cntP_ref, st_ref, cd_ref, cs_ref):
    cd = cntP_ref[0, 0, :, 0:1] + cntP_ref[1, 0, :, 0:1]
    cs = cntP_ref[0, 1, :, 0:1] + cntP_ref[1, 1, :, 0:1]
    cd_ref[...] = cd
    cs_ref[...] = cs
    _stats_update(st_ref, x_ref[...], cs)


def _tc_stats(x, cntP):
    return pl.pallas_call(
        _tc_stats_body,
        grid=(NB,),
        in_specs=[pl.BlockSpec((BR, D), lambda i: (i, 0)),
                  pl.BlockSpec((2, 2, BR, HC), lambda i: (0, 0, i, 0))],
        out_specs=[pl.BlockSpec((D, 8), lambda i: (0, 0)),
                   pl.BlockSpec((BR, 1), lambda i: (i, 0)),
                   pl.BlockSpec((BR, 1), lambda i: (i, 0))],
        out_shape=[jax.ShapeDtypeStruct((D, 8), jnp.float32),
                   jax.ShapeDtypeStruct((N, 1), jnp.float32),
                   jax.ShapeDtypeStruct((N, 1), jnp.float32)],
    )(x, cntP)


def _tc_msg_body(h_ref, st_ref, g_ref, b_ref, Wsrc_ref, bsrc_ref, y_ref):
    mu = st_ref[:, 0:1] * (1.0 / E)
    var = st_ref[:, 1:2] * (1.0 / E) - mu * mu
    scale = g_ref[...] * lax.rsqrt(var + 1e-5)
    shift = b_ref[...] - mu * scale
    Wsrc = Wsrc_ref[...]
    Wp = scale * Wsrc
    bp = lax.dot_general(shift, Wsrc, (((0,), (0,)), ((), ())),
                         precision=_PREC) + bsrc_ref[...]
    z = jnp.dot(h_ref[...], Wp, precision=_PREC) + bp
    y_ref[...] = z * 0.5 * (1.0 + lax.erf(z * 0.7071067811865476))


def _tc_msg(h, st, g, b, Wsrc, bsrc):
    return pl.pallas_call(
        _tc_msg_body,
        grid=(NB,),
        in_specs=[pl.BlockSpec((BR, D), lambda i: (i, 0)),
                  _FULL((D, 8)), _FULL((D, 1)), _FULL((D, 1)),
                  _FULL((D, H)), _FULL((1, H))],
        out_specs=pl.BlockSpec((BR, H), lambda i: (i, 0)),
        out_shape=jax.ShapeDtypeStruct((N, H), jnp.float32),
    )(h, st, g, b, Wsrc, bsrc)


def _tc_apply_body(apply_relu, with_stats, dout,
                   h_ref, aggP_ref, cd_ref, cs_ref,
                   Wfca_ref, Wfcb_ref, bfc_ref, Wdst_ref, bdst_ref,
                   *out_refs):
    h_in = h_ref[...]
    agg = aggP_ref[0, :, :] + aggP_ref[1, :, :]
    neigh = agg / jnp.maximum(cd_ref[...], 1.0)
    rst = (jnp.dot(h_in, Wfca_ref[...], precision=_PREC)
           + jnp.dot(neigh, Wfcb_ref[...], precision=_PREC) + bfc_ref[...])
    if apply_relu:
        rst = jnp.maximum(rst, 0.0)
    h = jnp.dot(h_in, Wdst_ref[...], precision=_PREC) + bdst_ref[...] + rst
    out_refs[0][...] = h
    if with_stats:
        _stats_update(out_refs[1], h, cs_ref[...])


def _tc_apply(apply_relu, with_stats, dout,
              h, aggP, cd, cs, Wfca, Wfcb, bfc, Wdst, bdst):
    out_shape = [jax.ShapeDtypeStruct((N, dout), jnp.float32)]
    out_specs = [pl.BlockSpec((BR, dout), lambda i: (i, 0))]
    if with_stats:
        out_shape.append(jax.ShapeDtypeStruct((D, 8), jnp.float32))
        out_specs.append(pl.BlockSpec((D, 8), lambda i: (0, 0)))
    res = pl.pallas_call(
        functools.partial(_tc_apply_body, apply_relu, with_stats, dout),
        grid=(NB,),
        in_specs=[pl.BlockSpec((BR, D), lambda i: (i, 0)),
                  pl.BlockSpec((2, BR, H), lambda i: (0, i, 0)),
                  pl.BlockSpec((BR, 1), lambda i: (i, 0)),
                  pl.BlockSpec((BR, 1), lambda i: (i, 0)),
                  _FULL((D, dout)), _FULL((H, dout)), _FULL((1, dout)),
                  _FULL((D, dout)), _FULL((1, dout))],
        out_specs=out_specs,
        out_shape=out_shape,
    )(h, aggP, cd, cs, Wfca, Wfcb, bfc, Wdst, bdst)
    return res


@jax.jit
def kernel(x, edge_index,
           bn_g1, bn_b1, Wsrc1, bsrc1, Wfc1, bfc1, Wdst1, bdst1,
           bn_g2, bn_b2, Wsrc2, bsrc2, Wfc2, bfc2, Wdst2, bdst2,
           bn_g3, bn_b3, Wsrc3, bsrc3, Wfc3, bfc3, Wdst3, bdst3):
    src = edge_index[0]
    dst = edge_index[1]
    pspread = jnp.arange(EPW_PAD - EPW, dtype=jnp.int32)[None, :] % 16
    spad = pspread
    tpad = TRASH + pspread
    spad = jnp.broadcast_to(spad, (NW, EPW_PAD - EPW))
    tpad = jnp.broadcast_to(tpad, (NW, EPW_PAD - EPW))
    srcs_g = jnp.concatenate([src.reshape(NW, EPW), spad], axis=1)
    srcs_g = srcs_g.reshape(NW, CH, CW)
    srcs_c = jnp.concatenate([src.reshape(NW, EPW), tpad], axis=1)
    srcs_c = srcs_c.reshape(NW, CH, CW)
    dsts_r = jnp.concatenate([dst.reshape(NW, EPW), tpad], axis=1)
    dsts_r = dsts_r.reshape(NW, CH, CW)

    zeros = jnp.zeros((NNP, H), jnp.float32)
    zeros_n = jnp.zeros((NNP, HC), jnp.float32)
    ones_rows = jnp.ones((CW, HC), jnp.float32)

    cntP = _sc_cnt(ones_rows, srcs_c, dsts_r, zeros_n)

    g1 = bn_g1.reshape(D, 1); b1 = bn_b1.reshape(D, 1)
    g2 = bn_g2.reshape(D, 1); b2 = bn_b2.reshape(D, 1)
    g3 = bn_g3.reshape(D, 1); b3 = bn_b3.reshape(D, 1)

    st1, cd, cs = _tc_stats(x, cntP)
    y1 = _tc_msg(x, st1, g1, b1, Wsrc1, bsrc1.reshape(1, H))
    aggP1 = _sc_agg(y1, srcs_g, dsts_r, zeros)
    h1, st2 = _tc_apply(True, True, D, x, aggP1, cd, cs,
                        Wfc1[0:D], Wfc1[D:], bfc1.reshape(1, D),
                        Wdst1, bdst1.reshape(1, D))
    y2 = _tc_msg(h1, st2, g2, b2, Wsrc2, bsrc2.reshape(1, H))
    aggP2 = _sc_agg(y2, srcs_g, dsts_r, zeros)
    h2, st3 = _tc_apply(False, True, D, h1, aggP2, cd, cs,
                        Wfc2[0:D], Wfc2[D:], bfc2.reshape(1, D),
                        Wdst2, bdst2.reshape(1, D))
    y3 = _tc_msg(h2, st3, g3, b3, Wsrc3, bsrc3.reshape(1, H))
    aggP3 = _sc_agg(y3, srcs_g, dsts_r, zeros)
    (out,) = _tc_apply(False, False, 1, h2, aggP3, cd, cs,
                       Wfc3[0:D], Wfc3[D:], bfc3.reshape(1, 1),
                       Wdst3, bdst3.reshape(1, 1))
    return out
